# Initial kernel scaffold; baseline (speedup 1.0000x reference)
import jax
import jax.numpy as jnp
from jax.experimental import pallas as pl


def kernel(times, data, t):
    raise NotImplementedError("write your pallas kernel here")


# speedup vs baseline: 1.3833x; 1.3833x over previous
import functools

import jax
import jax.numpy as jnp
from jax import lax
from jax.experimental import pallas as pl
from jax.experimental.pallas import tpu as pltpu
from jax.experimental.pallas import tpu_sc as plsc

NTIME = 1024
NBATCH = 16384
L = 16
NC = 2
NS = 16
CHUNK = NBATCH // (NC * NS)


def _interp_body(times_hbm, data_hbm, t_hbm, colidx_hbm, out_hbm,
                 idx_v, col_v, t0_v, t_v, d0_v, d1_v, x0_v, x1_v, out_v,
                 sem_g, sem_l):
    wid = lax.axis_index("c") * NS + lax.axis_index("s")
    base = wid * CHUNK

    pltpu.sync_copy(colidx_hbm, idx_v)
    gathers = [
        pltpu.async_copy(times_hbm.at[idx_v.at[j]],
                         col_v.at[pl.ds(j * 128, 128)], sem_g)
        for j in range(NTIME // 128)
    ]
    lin = [
        pltpu.async_copy(t_hbm.at[pl.ds(0, L)], t0_v, sem_l),
        pltpu.async_copy(t_hbm.at[pl.ds(base, CHUNK)], t_v, sem_l),
    ]
    for cp in gathers + lin:
        cp.wait()

    t0 = t0_v[...][0]

    ones = lax.iota(jnp.int32, L) * 0 + 1
    zeros = lax.iota(jnp.int32, L) * 0

    def count_body(i, acc):
        v = col_v[pl.ds(i * L, L)]
        return acc + jnp.where(v < t0, ones, zeros)

    acc = lax.fori_loop(0, NTIME // L, count_body, zeros)
    cnt = acc[0]
    for i in range(1, L):
        cnt = cnt + acc[i]
    gi = jnp.where(cnt >= NTIME, 1, jnp.maximum(cnt, 1))
    o0 = (gi - 1) * NBATCH + base
    o1 = gi * NBATCH + base

    rows = [
        pltpu.async_copy(data_hbm.at[pl.ds(o0, CHUNK)], d0_v, sem_l),
        pltpu.async_copy(data_hbm.at[pl.ds(o1, CHUNK)], d1_v, sem_l),
        pltpu.async_copy(times_hbm.at[pl.ds(o0, CHUNK)], x0_v, sem_l),
        pltpu.async_copy(times_hbm.at[pl.ds(o1, CHUNK)], x1_v, sem_l),
    ]
    for cp in rows:
        cp.wait()

    for i in range(CHUNK // L):
        s = pl.ds(i * L, L)
        d0 = d0_v[s]
        x0 = x0_v[s]
        slope = (d1_v[s] - d0) / (x1_v[s] - x0)
        out_v[s] = d0 + slope * (t_v[s] - x0)

    pltpu.sync_copy(out_v, out_hbm.at[pl.ds(base, CHUNK)])


def kernel(times, data, t):
    times1d = times.reshape(-1)
    data1d = data.reshape(-1)
    colidx = (jnp.arange(NTIME, dtype=jnp.int32) * NBATCH).reshape(-1, 128)

    mesh = plsc.VectorSubcoreMesh(core_axis_name="c", subcore_axis_name="s")
    f = functools.partial(
        pl.kernel,
        mesh=mesh,
        out_type=jax.ShapeDtypeStruct((NBATCH,), jnp.float32),
        scratch_types=[
            pltpu.VMEM((NTIME // 128, 128), jnp.int32),
            pltpu.VMEM((NTIME,), jnp.float32),
            pltpu.VMEM((L,), jnp.float32),
            pltpu.VMEM((CHUNK,), jnp.float32),
            pltpu.VMEM((CHUNK,), jnp.float32),
            pltpu.VMEM((CHUNK,), jnp.float32),
            pltpu.VMEM((CHUNK,), jnp.float32),
            pltpu.VMEM((CHUNK,), jnp.float32),
            pltpu.VMEM((CHUNK,), jnp.float32),
            pltpu.SemaphoreType.DMA,
            pltpu.SemaphoreType.DMA,
        ],
    )(_interp_body)
    return f(times1d, data1d, t, colidx)


# speedup vs baseline: 7.7783x; 5.6232x over previous
import functools

import jax
import jax.numpy as jnp
from jax import lax
from jax.experimental import pallas as pl
from jax.experimental.pallas import tpu as pltpu
from jax.experimental.pallas import tpu_sc as plsc

NTIME = 1024
NBATCH = 16384
L = 16
NC = 2
NS = 16
CHUNK = NBATCH // (NC * NS)


def _interp_body(times_hbm, data_hbm, tcol_hbm, t_hbm, out_hbm,
                 col_v, t0_v, t_v, d0_v, d1_v, x0_v, x1_v, out_v, sem):
    wid = lax.axis_index("c") * NS + lax.axis_index("s")
    base = wid * CHUNK

    lin = [
        pltpu.async_copy(tcol_hbm, col_v, sem),
        pltpu.async_copy(t_hbm.at[pl.ds(0, L)], t0_v, sem),
        pltpu.async_copy(t_hbm.at[pl.ds(base, CHUNK)], t_v, sem),
    ]
    for cp in lin:
        cp.wait()

    t0 = t0_v[...][0]

    ones = lax.iota(jnp.int32, L) * 0 + 1
    zeros = lax.iota(jnp.int32, L) * 0

    def count_body(i, acc):
        v = col_v[pl.ds(i * L, L)]
        return acc + jnp.where(v < t0, ones, zeros)

    acc = lax.fori_loop(0, NTIME // L, count_body, zeros)
    cnt = acc[0]
    for i in range(1, L):
        cnt = cnt + acc[i]
    gi = jnp.where(cnt >= NTIME, 1, jnp.maximum(cnt, 1))

    rows = [
        pltpu.async_copy(data_hbm.at[gi - 1, pl.ds(base, CHUNK)], d0_v, sem),
        pltpu.async_copy(data_hbm.at[gi, pl.ds(base, CHUNK)], d1_v, sem),
        pltpu.async_copy(times_hbm.at[gi - 1, pl.ds(base, CHUNK)], x0_v, sem),
        pltpu.async_copy(times_hbm.at[gi, pl.ds(base, CHUNK)], x1_v, sem),
    ]
    for cp in rows:
        cp.wait()

    for i in range(CHUNK // L):
        s = pl.ds(i * L, L)
        d0 = d0_v[s]
        x0 = x0_v[s]
        slope = (d1_v[s] - d0) / (x1_v[s] - x0)
        out_v[s] = d0 + slope * (t_v[s] - x0)

    pltpu.sync_copy(out_v, out_hbm.at[pl.ds(base, CHUNK)])


def kernel(times, data, t):
    tcol = times[:, 0]

    mesh = plsc.VectorSubcoreMesh(core_axis_name="c", subcore_axis_name="s")
    f = functools.partial(
        pl.kernel,
        mesh=mesh,
        out_type=jax.ShapeDtypeStruct((NBATCH,), jnp.float32),
        compiler_params=pltpu.CompilerParams(use_tc_tiling_on_sc=True),
        scratch_types=[
            pltpu.VMEM((NTIME,), jnp.float32),
            pltpu.VMEM((L,), jnp.float32),
            pltpu.VMEM((CHUNK,), jnp.float32),
            pltpu.VMEM((CHUNK,), jnp.float32),
            pltpu.VMEM((CHUNK,), jnp.float32),
            pltpu.VMEM((CHUNK,), jnp.float32),
            pltpu.VMEM((CHUNK,), jnp.float32),
            pltpu.VMEM((CHUNK,), jnp.float32),
            pltpu.SemaphoreType.DMA,
        ],
    )(_interp_body)
    return f(times, data, tcol, t)


# speedup vs baseline: 8.0307x; 1.0324x over previous
import functools

import jax
import jax.numpy as jnp
from jax import lax
from jax.experimental import pallas as pl
from jax.experimental.pallas import tpu as pltpu
from jax.experimental.pallas import tpu_sc as plsc

NTIME = 1024
NBATCH = 16384
L = 16
NC = 2
NS = 16
CHUNK = NBATCH // (NC * NS)


def _interp_body(times_hbm, data_hbm, tcol_hbm, t_hbm, out_hbm,
                 col_v, t0_v, t_v, d0_v, d1_v, x0_v, x1_v, out_v,
                 sem, sem_fix):
    wid = lax.axis_index("c") * NS + lax.axis_index("s")
    base = wid * CHUNK

    first = [
        pltpu.async_copy(tcol_hbm, col_v, sem),
        pltpu.async_copy(t_hbm.at[pl.ds(0, L)], t0_v, sem),
        pltpu.async_copy(t_hbm.at[pl.ds(base, CHUNK)], t_v, sem),
        pltpu.async_copy(data_hbm.at[0, pl.ds(base, CHUNK)], d0_v, sem),
        pltpu.async_copy(data_hbm.at[1, pl.ds(base, CHUNK)], d1_v, sem),
        pltpu.async_copy(times_hbm.at[0, pl.ds(base, CHUNK)], x0_v, sem),
        pltpu.async_copy(times_hbm.at[1, pl.ds(base, CHUNK)], x1_v, sem),
    ]
    for cp in first:
        cp.wait()

    t0 = t0_v[...][0]

    ones = lax.iota(jnp.int32, L) * 0 + 1
    zeros = lax.iota(jnp.int32, L) * 0
    acc = zeros
    for i in range(NTIME // L):
        v = col_v[pl.ds(i * L, L)]
        acc = acc + jnp.where(v < t0, ones, zeros)
    cnt = acc[0]
    for i in range(1, L):
        cnt = cnt + acc[i]
    gi = jnp.where(cnt >= NTIME, 1, jnp.maximum(cnt, 1))

    @pl.when(gi != 1)
    def _refetch():
        rows = [
            pltpu.async_copy(data_hbm.at[gi - 1, pl.ds(base, CHUNK)],
                             d0_v, sem_fix),
            pltpu.async_copy(data_hbm.at[gi, pl.ds(base, CHUNK)],
                             d1_v, sem_fix),
            pltpu.async_copy(times_hbm.at[gi - 1, pl.ds(base, CHUNK)],
                             x0_v, sem_fix),
            pltpu.async_copy(times_hbm.at[gi, pl.ds(base, CHUNK)],
                             x1_v, sem_fix),
        ]
        for cp in rows:
            cp.wait()

    for i in range(CHUNK // L):
        s = pl.ds(i * L, L)
        d0 = d0_v[s]
        x0 = x0_v[s]
        slope = (d1_v[s] - d0) / (x1_v[s] - x0)
        out_v[s] = d0 + slope * (t_v[s] - x0)

    pltpu.sync_copy(out_v, out_hbm.at[pl.ds(base, CHUNK)])


def kernel(times, data, t):
    tcol = times[:, 0]

    mesh = plsc.VectorSubcoreMesh(core_axis_name="c", subcore_axis_name="s")
    f = functools.partial(
        pl.kernel,
        mesh=mesh,
        out_type=jax.ShapeDtypeStruct((NBATCH,), jnp.float32),
        compiler_params=pltpu.CompilerParams(use_tc_tiling_on_sc=True),
        scratch_types=[
            pltpu.VMEM((NTIME,), jnp.float32),
            pltpu.VMEM((L,), jnp.float32),
            pltpu.VMEM((CHUNK,), jnp.float32),
            pltpu.VMEM((CHUNK,), jnp.float32),
            pltpu.VMEM((CHUNK,), jnp.float32),
            pltpu.VMEM((CHUNK,), jnp.float32),
            pltpu.VMEM((CHUNK,), jnp.float32),
            pltpu.VMEM((CHUNK,), jnp.float32),
            pltpu.SemaphoreType.DMA,
            pltpu.SemaphoreType.DMA,
        ],
    )(_interp_body)
    return f(times, data, tcol, t)


# speedup vs baseline: 8.0388x; 1.0010x over previous
import functools

import jax
import jax.numpy as jnp
from jax import lax
from jax.experimental import pallas as pl
from jax.experimental.pallas import tpu as pltpu
from jax.experimental.pallas import tpu_sc as plsc

NTIME = 1024
NBATCH = 16384
L = 16
NC = 2
NS = 16
CHUNK = NBATCH // (NC * NS)


def _interp_body(times_hbm, data_hbm, tcol_hbm, t_hbm, out_hbm,
                 col_v, t0_v, t_v, d0_v, d1_v, x0_v, x1_v, out_v,
                 sem, sem_fix):
    wid = lax.axis_index("c") * NS + lax.axis_index("s")
    base = wid * CHUNK

    first = [
        pltpu.async_copy(tcol_hbm, col_v, sem),
        pltpu.async_copy(t_hbm.at[pl.ds(0, L)], t0_v, sem),
        pltpu.async_copy(t_hbm.at[pl.ds(base, CHUNK)], t_v, sem),
        pltpu.async_copy(data_hbm.at[0, pl.ds(base, CHUNK)], d0_v, sem),
        pltpu.async_copy(data_hbm.at[1, pl.ds(base, CHUNK)], d1_v, sem),
        pltpu.async_copy(times_hbm.at[0, pl.ds(base, CHUNK)], x0_v, sem),
        pltpu.async_copy(times_hbm.at[1, pl.ds(base, CHUNK)], x1_v, sem),
    ]
    for cp in first:
        cp.wait()

    t0 = t0_v[...][0]

    ones = lax.iota(jnp.int32, L) * 0 + 1
    zeros = lax.iota(jnp.int32, L) * 0
    acc = zeros
    for i in range(NTIME // L):
        v = col_v[pl.ds(i * L, L)]
        acc = acc + jnp.where(v < t0, ones, zeros)
    cnt = acc[0]
    for i in range(1, L):
        cnt = cnt + acc[i]
    gi = jnp.where(cnt >= NTIME, 1, jnp.maximum(cnt, 1))

    @pl.when(gi != 1)
    def _refetch():
        rows = [
            pltpu.async_copy(data_hbm.at[gi - 1, pl.ds(base, CHUNK)],
                             d0_v, sem_fix),
            pltpu.async_copy(data_hbm.at[gi, pl.ds(base, CHUNK)],
                             d1_v, sem_fix),
            pltpu.async_copy(times_hbm.at[gi - 1, pl.ds(base, CHUNK)],
                             x0_v, sem_fix),
            pltpu.async_copy(times_hbm.at[gi, pl.ds(base, CHUNK)],
                             x1_v, sem_fix),
        ]
        for cp in rows:
            cp.wait()

    for i in range(CHUNK // L):
        s = pl.ds(i * L, L)
        d0 = d0_v[s]
        x0 = x0_v[s]
        slope = (d1_v[s] - d0) / (x1_v[s] - x0)
        out_v[s] = d0 + slope * (t_v[s] - x0)

    pltpu.sync_copy(out_v, out_hbm.at[pl.ds(base, CHUNK)])


def kernel(times, data, t):
    tcol = times[:, 0]

    mesh = plsc.VectorSubcoreMesh(core_axis_name="c", subcore_axis_name="s")
    f = functools.partial(
        pl.kernel,
        mesh=mesh,
        out_type=jax.ShapeDtypeStruct((NBATCH,), jnp.float32),
        compiler_params=pltpu.CompilerParams(use_tc_tiling_on_sc=True,
                                             skip_device_barrier=True),
        scratch_types=[
            pltpu.VMEM((NTIME,), jnp.float32),
            pltpu.VMEM((L,), jnp.float32),
            pltpu.VMEM((CHUNK,), jnp.float32),
            pltpu.VMEM((CHUNK,), jnp.float32),
            pltpu.VMEM((CHUNK,), jnp.float32),
            pltpu.VMEM((CHUNK,), jnp.float32),
            pltpu.VMEM((CHUNK,), jnp.float32),
            pltpu.VMEM((CHUNK,), jnp.float32),
            pltpu.SemaphoreType.DMA,
            pltpu.SemaphoreType.DMA,
        ],
    )(_interp_body)
    return f(times, data, tcol, t)


# speedup vs baseline: 8.2845x; 1.0306x over previous
import functools

import jax
import jax.numpy as jnp
from jax import lax
from jax.experimental import pallas as pl
from jax.experimental.pallas import tpu as pltpu
from jax.experimental.pallas import tpu_sc as plsc

NTIME = 1024
NBATCH = 16384
L = 16
NC = 2
NS = 16
CHUNK = NBATCH // NS


def _interp_body(times_hbm, data_hbm, tcol_hbm, t_hbm, out_hbm,
                 col_v, t0_v, t_v, d0_v, d1_v, x0_v, x1_v, out_v,
                 sem, sem_fix):
    wid = lax.axis_index("s")
    base = wid * CHUNK

    first = [
        pltpu.async_copy(tcol_hbm, col_v, sem),
        pltpu.async_copy(t_hbm.at[pl.ds(0, L)], t0_v, sem),
        pltpu.async_copy(t_hbm.at[pl.ds(base, CHUNK)], t_v, sem),
        pltpu.async_copy(data_hbm.at[0, pl.ds(base, CHUNK)], d0_v, sem),
        pltpu.async_copy(data_hbm.at[1, pl.ds(base, CHUNK)], d1_v, sem),
        pltpu.async_copy(times_hbm.at[0, pl.ds(base, CHUNK)], x0_v, sem),
        pltpu.async_copy(times_hbm.at[1, pl.ds(base, CHUNK)], x1_v, sem),
    ]
    for cp in first:
        cp.wait()

    t0 = t0_v[...][0]

    ones = lax.iota(jnp.int32, L) * 0 + 1
    zeros = lax.iota(jnp.int32, L) * 0
    acc = zeros
    for i in range(NTIME // L):
        v = col_v[pl.ds(i * L, L)]
        acc = acc + jnp.where(v < t0, ones, zeros)
    cnt = acc[0]
    for i in range(1, L):
        cnt = cnt + acc[i]
    gi = jnp.where(cnt >= NTIME, 1, jnp.maximum(cnt, 1))

    @pl.when(gi != 1)
    def _refetch():
        rows = [
            pltpu.async_copy(data_hbm.at[gi - 1, pl.ds(base, CHUNK)],
                             d0_v, sem_fix),
            pltpu.async_copy(data_hbm.at[gi, pl.ds(base, CHUNK)],
                             d1_v, sem_fix),
            pltpu.async_copy(times_hbm.at[gi - 1, pl.ds(base, CHUNK)],
                             x0_v, sem_fix),
            pltpu.async_copy(times_hbm.at[gi, pl.ds(base, CHUNK)],
                             x1_v, sem_fix),
        ]
        for cp in rows:
            cp.wait()

    for i in range(CHUNK // L):
        s = pl.ds(i * L, L)
        d0 = d0_v[s]
        x0 = x0_v[s]
        slope = (d1_v[s] - d0) / (x1_v[s] - x0)
        out_v[s] = d0 + slope * (t_v[s] - x0)

    pltpu.sync_copy(out_v, out_hbm.at[pl.ds(base, CHUNK)])


def kernel(times, data, t):
    tcol = times[:, 0]

    mesh = plsc.VectorSubcoreMesh(core_axis_name="c", subcore_axis_name="s", num_cores=1)
    f = functools.partial(
        pl.kernel,
        mesh=mesh,
        out_type=jax.ShapeDtypeStruct((NBATCH,), jnp.float32),
        compiler_params=pltpu.CompilerParams(use_tc_tiling_on_sc=True,
                                             skip_device_barrier=True),
        scratch_types=[
            pltpu.VMEM((NTIME,), jnp.float32),
            pltpu.VMEM((L,), jnp.float32),
            pltpu.VMEM((CHUNK,), jnp.float32),
            pltpu.VMEM((CHUNK,), jnp.float32),
            pltpu.VMEM((CHUNK,), jnp.float32),
            pltpu.VMEM((CHUNK,), jnp.float32),
            pltpu.VMEM((CHUNK,), jnp.float32),
            pltpu.VMEM((CHUNK,), jnp.float32),
            pltpu.SemaphoreType.DMA,
            pltpu.SemaphoreType.DMA,
        ],
    )(_interp_body)
    return f(times, data, tcol, t)


# speedup vs baseline: 8.4440x; 1.0192x over previous
import functools

import jax
import jax.numpy as jnp
from jax import lax
from jax.experimental import pallas as pl
from jax.experimental.pallas import tpu as pltpu
from jax.experimental.pallas import tpu_sc as plsc

NTIME = 1024
NBATCH = 16384
L = 16
NC = 2
NS = 16
CHUNK = NBATCH // NS


def _interp_body(times_hbm, data_hbm, tcol_hbm, t_hbm, out_hbm,
                 col_v, t0_v, t_v, d0_v, d1_v, x0_v, x1_v, out_v,
                 sem, sem_fix):
    wid = lax.axis_index("s")
    base = wid * CHUNK

    first = [
        pltpu.async_copy(tcol_hbm, col_v, sem),
        pltpu.async_copy(t_hbm.at[pl.ds(0, L)], t0_v, sem),
        pltpu.async_copy(t_hbm.at[pl.ds(base, CHUNK)], t_v, sem),
        pltpu.async_copy(data_hbm.at[0, pl.ds(base, CHUNK)], d0_v, sem),
        pltpu.async_copy(data_hbm.at[1, pl.ds(base, CHUNK)], d1_v, sem),
        pltpu.async_copy(times_hbm.at[0, pl.ds(base, CHUNK)], x0_v, sem),
        pltpu.async_copy(times_hbm.at[1, pl.ds(base, CHUNK)], x1_v, sem),
    ]
    for cp in first:
        cp.wait()

    t0 = t0_v[...][0]

    ones = lax.iota(jnp.int32, L) * 0 + 1
    zeros = lax.iota(jnp.int32, L) * 0
    acc = zeros
    for i in range(NTIME // L):
        v = col_v[pl.ds(i * L, L)]
        acc = acc + jnp.where(v < t0, ones, zeros)
    cnt = acc[0]
    for i in range(1, L):
        cnt = cnt + acc[i]
    gi = jnp.where(cnt >= NTIME, 1, jnp.maximum(cnt, 1))

    @pl.when(gi != 1)
    def _refetch():
        rows = [
            pltpu.async_copy(data_hbm.at[gi - 1, pl.ds(base, CHUNK)],
                             d0_v, sem_fix),
            pltpu.async_copy(data_hbm.at[gi, pl.ds(base, CHUNK)],
                             d1_v, sem_fix),
            pltpu.async_copy(times_hbm.at[gi - 1, pl.ds(base, CHUNK)],
                             x0_v, sem_fix),
            pltpu.async_copy(times_hbm.at[gi, pl.ds(base, CHUNK)],
                             x1_v, sem_fix),
        ]
        for cp in rows:
            cp.wait()

    half = CHUNK // 2
    stores = []
    for h in range(2):
        for i in range(h * (half // L), (h + 1) * (half // L)):
            s = pl.ds(i * L, L)
            d0 = d0_v[s]
            x0 = x0_v[s]
            slope = (d1_v[s] - d0) / (x1_v[s] - x0)
            out_v[s] = d0 + slope * (t_v[s] - x0)
        stores.append(pltpu.async_copy(
            out_v.at[pl.ds(h * half, half)],
            out_hbm.at[pl.ds(base + h * half, half)], sem))
    for cp in stores:
        cp.wait()


def kernel(times, data, t):
    tcol = times[:, 0]

    mesh = plsc.VectorSubcoreMesh(core_axis_name="c", subcore_axis_name="s", num_cores=1)
    f = functools.partial(
        pl.kernel,
        mesh=mesh,
        out_type=jax.ShapeDtypeStruct((NBATCH,), jnp.float32),
        compiler_params=pltpu.CompilerParams(use_tc_tiling_on_sc=True,
                                             skip_device_barrier=True),
        scratch_types=[
            pltpu.VMEM((NTIME,), jnp.float32),
            pltpu.VMEM((L,), jnp.float32),
            pltpu.VMEM((CHUNK,), jnp.float32),
            pltpu.VMEM((CHUNK,), jnp.float32),
            pltpu.VMEM((CHUNK,), jnp.float32),
            pltpu.VMEM((CHUNK,), jnp.float32),
            pltpu.VMEM((CHUNK,), jnp.float32),
            pltpu.VMEM((CHUNK,), jnp.float32),
            pltpu.SemaphoreType.DMA,
            pltpu.SemaphoreType.DMA,
        ],
    )(_interp_body)
    return f(times, data, tcol, t)


# speedup vs baseline: 8.4581x; 1.0017x over previous
import functools

import jax
import jax.numpy as jnp
from jax import lax
from jax.experimental import pallas as pl
from jax.experimental.pallas import tpu as pltpu
from jax.experimental.pallas import tpu_sc as plsc

NTIME = 1024
NBATCH = 16384
L = 16
NC = 2
NS = 16
CHUNK = NBATCH // NS


def _interp_body(times_hbm, data_hbm, tcol_hbm, t_hbm, out_hbm,
                 col_v, t0_v, t_v, d0_v, d1_v, x0_v, x1_v, out_v,
                 sem, sem_fix, sem_c0, sem_c1, sem_t0):
    wid = lax.axis_index("s")
    base = wid * CHUNK

    ch = NTIME // 2
    cp_c0 = pltpu.async_copy(tcol_hbm.at[pl.ds(0, ch)],
                             col_v.at[pl.ds(0, ch)], sem_c0)
    cp_c1 = pltpu.async_copy(tcol_hbm.at[pl.ds(ch, ch)],
                             col_v.at[pl.ds(ch, ch)], sem_c1)
    cp_t0 = pltpu.async_copy(t_hbm.at[pl.ds(0, L)], t0_v, sem_t0)
    bulk = [
        pltpu.async_copy(t_hbm.at[pl.ds(base, CHUNK)], t_v, sem),
        pltpu.async_copy(data_hbm.at[0, pl.ds(base, CHUNK)], d0_v, sem),
        pltpu.async_copy(data_hbm.at[1, pl.ds(base, CHUNK)], d1_v, sem),
        pltpu.async_copy(times_hbm.at[0, pl.ds(base, CHUNK)], x0_v, sem),
        pltpu.async_copy(times_hbm.at[1, pl.ds(base, CHUNK)], x1_v, sem),
    ]

    cp_t0.wait()
    t0 = t0_v[...][0]

    ones = lax.iota(jnp.int32, L) * 0 + 1
    zeros = lax.iota(jnp.int32, L) * 0
    acc = zeros
    cp_c0.wait()
    for i in range(ch // L):
        v = col_v[pl.ds(i * L, L)]
        acc = acc + jnp.where(v < t0, ones, zeros)
    cp_c1.wait()
    for i in range(ch // L, NTIME // L):
        v = col_v[pl.ds(i * L, L)]
        acc = acc + jnp.where(v < t0, ones, zeros)
    cnt = acc[0]
    for i in range(1, L):
        cnt = cnt + acc[i]
    gi = jnp.where(cnt >= NTIME, 1, jnp.maximum(cnt, 1))

    for cp in bulk:
        cp.wait()

    @pl.when(gi != 1)
    def _refetch():
        rows = [
            pltpu.async_copy(data_hbm.at[gi - 1, pl.ds(base, CHUNK)],
                             d0_v, sem_fix),
            pltpu.async_copy(data_hbm.at[gi, pl.ds(base, CHUNK)],
                             d1_v, sem_fix),
            pltpu.async_copy(times_hbm.at[gi - 1, pl.ds(base, CHUNK)],
                             x0_v, sem_fix),
            pltpu.async_copy(times_hbm.at[gi, pl.ds(base, CHUNK)],
                             x1_v, sem_fix),
        ]
        for cp in rows:
            cp.wait()

    half = CHUNK // 2
    stores = []
    for h in range(2):
        for i in range(h * (half // L), (h + 1) * (half // L)):
            s = pl.ds(i * L, L)
            d0 = d0_v[s]
            x0 = x0_v[s]
            slope = (d1_v[s] - d0) / (x1_v[s] - x0)
            out_v[s] = d0 + slope * (t_v[s] - x0)
        stores.append(pltpu.async_copy(
            out_v.at[pl.ds(h * half, half)],
            out_hbm.at[pl.ds(base + h * half, half)], sem))
    for cp in stores:
        cp.wait()


def kernel(times, data, t):
    tcol = times[:, 0]

    mesh = plsc.VectorSubcoreMesh(core_axis_name="c", subcore_axis_name="s", num_cores=1)
    f = functools.partial(
        pl.kernel,
        mesh=mesh,
        out_type=jax.ShapeDtypeStruct((NBATCH,), jnp.float32),
        compiler_params=pltpu.CompilerParams(use_tc_tiling_on_sc=True,
                                             skip_device_barrier=True),
        scratch_types=[
            pltpu.VMEM((NTIME,), jnp.float32),
            pltpu.VMEM((L,), jnp.float32),
            pltpu.VMEM((CHUNK,), jnp.float32),
            pltpu.VMEM((CHUNK,), jnp.float32),
            pltpu.VMEM((CHUNK,), jnp.float32),
            pltpu.VMEM((CHUNK,), jnp.float32),
            pltpu.VMEM((CHUNK,), jnp.float32),
            pltpu.VMEM((CHUNK,), jnp.float32),
            pltpu.SemaphoreType.DMA,
            pltpu.SemaphoreType.DMA,
            pltpu.SemaphoreType.DMA,
            pltpu.SemaphoreType.DMA,
            pltpu.SemaphoreType.DMA,
        ],
    )(_interp_body)
    return f(times, data, tcol, t)


# speedup vs baseline: 8.4655x; 1.0009x over previous
import functools

import jax
import jax.numpy as jnp
from jax import lax
from jax.experimental import pallas as pl
from jax.experimental.pallas import tpu as pltpu
from jax.experimental.pallas import tpu_sc as plsc

NTIME = 1024
NBATCH = 16384
L = 16
NS = 16
CHUNK = NBATCH // NS


def _interp_body(times_hbm, data_hbm, tcol_hbm, t_hbm, out_hbm,
                 col_v, t0_v, t_v, d0_v, d1_v, x0_v, x1_v, out_v,
                 sem, sem_fix, sem_c0, sem_c1, sem_t0):
    wid = lax.axis_index("s")
    base = wid * CHUNK

    ch = NTIME // 2
    cp_c0 = pltpu.async_copy(tcol_hbm.at[pl.ds(0, ch)],
                             col_v.at[pl.ds(0, ch)], sem_c0)
    cp_c1 = pltpu.async_copy(tcol_hbm.at[pl.ds(ch, ch)],
                             col_v.at[pl.ds(ch, ch)], sem_c1)
    cp_t0 = pltpu.async_copy(t_hbm.at[pl.ds(0, L)], t0_v, sem_t0)
    bulk = [
        pltpu.async_copy(t_hbm.at[pl.ds(base, CHUNK)], t_v, sem),
        pltpu.async_copy(data_hbm.at[0, pl.ds(base, CHUNK)], d0_v, sem),
        pltpu.async_copy(data_hbm.at[1, pl.ds(base, CHUNK)], d1_v, sem),
        pltpu.async_copy(times_hbm.at[0, pl.ds(base, CHUNK)], x0_v, sem),
        pltpu.async_copy(times_hbm.at[1, pl.ds(base, CHUNK)], x1_v, sem),
    ]

    cp_t0.wait()
    t0 = t0_v[...][0]

    ones = lax.iota(jnp.int32, L) * 0 + 1
    zeros = lax.iota(jnp.int32, L) * 0
    acc = zeros
    cp_c0.wait()
    for i in range(ch // L):
        v = col_v[pl.ds(i * L, L)]
        acc = acc + jnp.where(v < t0, ones, zeros)
    cp_c1.wait()
    for i in range(ch // L, NTIME // L):
        v = col_v[pl.ds(i * L, L)]
        acc = acc + jnp.where(v < t0, ones, zeros)
    cnt = acc[0]
    for i in range(1, L):
        cnt = cnt + acc[i]
    gi = jnp.where(cnt >= NTIME, 1, jnp.maximum(cnt, 1))

    for cp in bulk:
        cp.wait()

    @pl.when(gi != 1)
    def _refetch():
        rows = [
            pltpu.async_copy(data_hbm.at[gi - 1, pl.ds(base, CHUNK)],
                             d0_v, sem_fix),
            pltpu.async_copy(data_hbm.at[gi, pl.ds(base, CHUNK)],
                             d1_v, sem_fix),
            pltpu.async_copy(times_hbm.at[gi - 1, pl.ds(base, CHUNK)],
                             x0_v, sem_fix),
            pltpu.async_copy(times_hbm.at[gi, pl.ds(base, CHUNK)],
                             x1_v, sem_fix),
        ]
        for cp in rows:
            cp.wait()

    half = CHUNK // 2
    stores = []
    for h in range(2):
        for i in range(h * (half // L), (h + 1) * (half // L)):
            s = pl.ds(i * L, L)
            d0 = d0_v[s]
            x0 = x0_v[s]
            slope = (d1_v[s] - d0) / (x1_v[s] - x0)
            out_v[s] = d0 + slope * (t_v[s] - x0)
        stores.append(pltpu.async_copy(
            out_v.at[pl.ds(h * half, half)],
            out_hbm.at[pl.ds(base + h * half, half)], sem))
    for cp in stores:
        cp.wait()


def kernel(times, data, t):
    tcol = times[:, 0]

    mesh = plsc.VectorSubcoreMesh(core_axis_name="c", subcore_axis_name="s", num_cores=1)
    f = functools.partial(
        pl.kernel,
        mesh=mesh,
        out_type=jax.ShapeDtypeStruct((NBATCH,), jnp.float32),
        compiler_params=pltpu.CompilerParams(use_tc_tiling_on_sc=True,
                                             skip_device_barrier=True),
        scratch_types=[
            pltpu.VMEM((NTIME,), jnp.float32),
            pltpu.VMEM((L,), jnp.float32),
            pltpu.VMEM((CHUNK,), jnp.float32),
            pltpu.VMEM((CHUNK,), jnp.float32),
            pltpu.VMEM((CHUNK,), jnp.float32),
            pltpu.VMEM((CHUNK,), jnp.float32),
            pltpu.VMEM((CHUNK,), jnp.float32),
            pltpu.VMEM((CHUNK,), jnp.float32),
            pltpu.SemaphoreType.DMA,
            pltpu.SemaphoreType.DMA,
            pltpu.SemaphoreType.DMA,
            pltpu.SemaphoreType.DMA,
            pltpu.SemaphoreType.DMA,
        ],
    )(_interp_body)
    return f(times, data, tcol, t)
